# baseline (speedup 1.0000x reference)
import numpy as np
import jax
import jax.numpy as jnp
from jax.experimental import pallas as pl

_LEVELS = np.array([8, 8, 8, 5, 5, 5], dtype=np.int32)
_D = int(_LEVELS.shape[0])
_EPS = 0.001


def _scalar_consts(level: int):
    lf = float(level)
    half_l = (lf - 1.0) * (1.0 + _EPS) / 2.0
    offset = 0.5 if level % 2 == 0 else 0.0
    shift = float(np.arctanh(offset / half_l))
    inv_hw = 1.0 / float(level // 2)
    return half_l, offset, shift, inv_hw


_HL8, _OF8, _SH8, _IH8 = _scalar_consts(8)
_HL5, _OF5, _SH5, _IH5 = _scalar_consts(5)


def _fsq_body(z_ref, wc_ref, bc_ref, we_ref, be_ref, o_ref):
    z = z_ref[...]
    wc = wc_ref[...]
    zc = jax.lax.dot_general(
        z, wc, (((1,), (1,)), ((), ())), preferred_element_type=jnp.float32
    )
    zc = zc + bc_ref[...]
    lane = jax.lax.broadcasted_iota(jnp.int32, zc.shape, 1)
    is8 = lane < 3
    half_l = jnp.where(is8, _HL8, _HL5)
    offset = jnp.where(is8, _OF8, _OF5)
    shift = jnp.where(is8, _SH8, _SH5)
    inv_hw = jnp.where(is8, _IH8, _IH5)
    bounded = jnp.tanh(zc + shift) * half_l - offset
    codes = jnp.round(bounded) * inv_hw
    zq = jax.lax.dot_general(
        codes, we_ref[...], (((1,), (1,)), ((), ())),
        preferred_element_type=jnp.float32,
    )
    o_ref[...] = zq + be_ref[...]


def kernel(z, Wc, bc, We, be):
    B, H, C = z.shape
    N = B * H
    zf = z.reshape(N, C)
    bcr = bc.reshape(1, _D)
    ber = be.reshape(1, C)

    tb = 2304
    out = pl.pallas_call(
        _fsq_body,
        grid=(N // tb,),
        in_specs=[
            pl.BlockSpec((tb, C), lambda i: (i, 0)),
            pl.BlockSpec((_D, C), lambda i: (0, 0)),
            pl.BlockSpec((1, _D), lambda i: (0, 0)),
            pl.BlockSpec((C, _D), lambda i: (0, 0)),
            pl.BlockSpec((1, C), lambda i: (0, 0)),
        ],
        out_specs=pl.BlockSpec((tb, C), lambda i: (i, 0)),
        out_shape=jax.ShapeDtypeStruct((N, C), jnp.float32),
    )(zf, Wc, bcr, We, ber)

    return out.reshape(B, H, C), jnp.array(0.0, dtype=jnp.float32)


# speedup vs baseline: 1.0014x; 1.0014x over previous
import numpy as np
import jax
import jax.numpy as jnp
from jax.experimental import pallas as pl

_LEVELS = np.array([8, 8, 8, 5, 5, 5], dtype=np.int32)
_D = int(_LEVELS.shape[0])
_EPS = 0.001


def _scalar_consts(level: int):
    lf = float(level)
    half_l = (lf - 1.0) * (1.0 + _EPS) / 2.0
    offset = 0.5 if level % 2 == 0 else 0.0
    shift = float(np.arctanh(offset / half_l))
    inv_hw = 1.0 / float(level // 2)
    return half_l, offset, shift, inv_hw


_HL8, _OF8, _SH8, _IH8 = _scalar_consts(8)
_HL5, _OF5, _SH5, _IH5 = _scalar_consts(5)


def _fsq_body(z_ref, wc_ref, bc_ref, we_ref, be_ref, o_ref):
    z = z_ref[...]
    wc = wc_ref[...]
    zc = jax.lax.dot_general(
        z, wc, (((1,), (1,)), ((), ())), preferred_element_type=jnp.float32
    )
    zc = zc + bc_ref[...]
    lane = jax.lax.broadcasted_iota(jnp.int32, zc.shape, 1)
    is8 = lane < 3
    half_l = jnp.where(is8, _HL8, _HL5)
    offset = jnp.where(is8, _OF8, _OF5)
    shift = jnp.where(is8, _SH8, _SH5)
    inv_hw = jnp.where(is8, _IH8, _IH5)
    bounded = jnp.tanh(zc + shift) * half_l - offset
    codes = jnp.round(bounded) * inv_hw
    zq = jax.lax.dot_general(
        codes, we_ref[...], (((1,), (1,)), ((), ())),
        preferred_element_type=jnp.float32,
    )
    o_ref[...] = zq + be_ref[...]


def kernel(z, Wc, bc, We, be):
    B, H, C = z.shape
    N = B * H
    zf = z.reshape(N, C)
    bcr = bc.reshape(1, _D)
    ber = be.reshape(1, C)

    tb = 3072
    out = pl.pallas_call(
        _fsq_body,
        grid=(N // tb,),
        in_specs=[
            pl.BlockSpec((tb, C), lambda i: (i, 0)),
            pl.BlockSpec((_D, C), lambda i: (0, 0)),
            pl.BlockSpec((1, _D), lambda i: (0, 0)),
            pl.BlockSpec((C, _D), lambda i: (0, 0)),
            pl.BlockSpec((1, C), lambda i: (0, 0)),
        ],
        out_specs=pl.BlockSpec((tb, C), lambda i: (i, 0)),
        out_shape=jax.ShapeDtypeStruct((N, C), jnp.float32),
    )(zf, Wc, bcr, We, ber)

    return out.reshape(B, H, C), jnp.array(0.0, dtype=jnp.float32)


# speedup vs baseline: 1.0149x; 1.0135x over previous
import numpy as np
import jax
import jax.numpy as jnp
from jax.experimental import pallas as pl

_LEVELS = np.array([8, 8, 8, 5, 5, 5], dtype=np.int32)
_D = int(_LEVELS.shape[0])
_EPS = 0.001


def _scalar_consts(level: int):
    lf = float(level)
    half_l = (lf - 1.0) * (1.0 + _EPS) / 2.0
    offset = 0.5 if level % 2 == 0 else 0.0
    shift = float(np.arctanh(offset / half_l))
    inv_hw = 1.0 / float(level // 2)
    return half_l, offset, shift, inv_hw


_HL8, _OF8, _SH8, _IH8 = _scalar_consts(8)
_HL5, _OF5, _SH5, _IH5 = _scalar_consts(5)


def _fsq_body(z_ref, wc_ref, bc_ref, we_ref, be_ref, o_ref):
    z = z_ref[...]
    wc = wc_ref[...]
    zc = jax.lax.dot_general(
        z, wc, (((1,), (1,)), ((), ())), preferred_element_type=jnp.float32
    )
    zc = zc + bc_ref[...]
    lane = jax.lax.broadcasted_iota(jnp.int32, zc.shape, 1)
    is8 = lane < 3
    half_l = jnp.where(is8, _HL8, _HL5)
    offset = jnp.where(is8, _OF8, _OF5)
    shift = jnp.where(is8, _SH8, _SH5)
    inv_hw = jnp.where(is8, _IH8, _IH5)
    bounded = jnp.tanh(zc + shift) * half_l - offset
    codes = (jnp.round(bounded) * inv_hw).astype(jnp.bfloat16)
    zq = jax.lax.dot_general(
        codes, we_ref[...].astype(jnp.bfloat16), (((1,), (1,)), ((), ())),
        preferred_element_type=jnp.float32,
    )
    o_ref[...] = zq + be_ref[...]


def kernel(z, Wc, bc, We, be):
    B, H, C = z.shape
    N = B * H
    zf = z.reshape(N, C)
    bcr = bc.reshape(1, _D)
    ber = be.reshape(1, C)

    tb = 4608
    out = pl.pallas_call(
        _fsq_body,
        grid=(N // tb,),
        in_specs=[
            pl.BlockSpec((tb, C), lambda i: (i, 0)),
            pl.BlockSpec((_D, C), lambda i: (0, 0)),
            pl.BlockSpec((1, _D), lambda i: (0, 0)),
            pl.BlockSpec((C, _D), lambda i: (0, 0)),
            pl.BlockSpec((1, C), lambda i: (0, 0)),
        ],
        out_specs=pl.BlockSpec((tb, C), lambda i: (i, 0)),
        out_shape=jax.ShapeDtypeStruct((N, C), jnp.float32),
    )(zf, Wc, bcr, We, ber)

    return out.reshape(B, H, C), jnp.array(0.0, dtype=jnp.float32)
